# baseline (device time: 26651 ns/iter reference)
import jax
import jax.numpy as jnp
from jax import lax
from jax.experimental import pallas as pl
from jax.experimental.pallas import tpu as pltpu

M = 512
HALF = M // 2
C = 8


def kernel(dy, W):
    m, k = dy.shape
    n = W.shape[0]
    cw = n // C

    def body(dy_ref, w_ref, out_ref, p_ref, q_ref, r1_ref, r2_ref,
             s1, v1, s2, v2):
        my_x = lax.axis_index("x")
        my_y = lax.axis_index("y")
        y_nbr = (my_x, 1 - my_y)
        x_nbr = (1 - my_x, my_y)

        def rdma1(c):
            return pltpu.make_async_remote_copy(
                src_ref=p_ref.at[c], dst_ref=r1_ref.at[c],
                send_sem=s1.at[c], recv_sem=v1.at[c],
                device_id=y_nbr, device_id_type=pl.DeviceIdType.MESH,
            )

        def rdma2(c):
            return pltpu.make_async_remote_copy(
                src_ref=q_ref.at[c], dst_ref=r2_ref.at[c],
                send_sem=s2.at[c], recv_sem=v2.at[c],
                device_id=x_nbr, device_id_type=pl.DeviceIdType.MESH,
            )

        barrier_sem = pltpu.get_barrier_semaphore()
        for nbr in (y_nbr, x_nbr):
            pl.semaphore_signal(
                barrier_sem, inc=1,
                device_id=nbr, device_id_type=pl.DeviceIdType.MESH,
            )

        row0 = my_x * HALF
        dy_rows = dy_ref[pl.ds(row0, HALF), :]
        for c in range(C):
            p_ref[c] = lax.dot_general(
                dy_rows, w_ref[c * cw:(c + 1) * cw, :],
                dimension_numbers=(((1,), (1,)), ((), ())),
                preferred_element_type=jnp.float32,
            )

        pl.semaphore_wait(barrier_sem, 2)

        for c in range(C):
            rdma1(c).start()

        for c in range(C):
            rdma1(c).wait_recv()
            q_ref[c] = p_ref[c] + r1_ref[c]
            rdma2(c).start()
            out_ref[pl.ds(row0, HALF), c * cw:(c + 1) * cw] = q_ref[c]

        for c in range(C):
            rdma2(c).wait_recv()
            out_ref[pl.ds((1 - my_x) * HALF, HALF), c * cw:(c + 1) * cw] = (
                r2_ref[c]
            )

        for c in range(C):
            rdma1(c).wait_send()
            rdma2(c).wait_send()

    return pl.pallas_call(
        body,
        out_shape=jax.ShapeDtypeStruct((m, n), jnp.float32),
        in_specs=[
            pl.BlockSpec(memory_space=pltpu.VMEM),
            pl.BlockSpec(memory_space=pltpu.VMEM),
        ],
        out_specs=pl.BlockSpec(memory_space=pltpu.VMEM),
        scratch_shapes=[
            pltpu.VMEM((C, HALF, cw), jnp.float32),
            pltpu.VMEM((C, HALF, cw), jnp.float32),
            pltpu.VMEM((C, HALF, cw), jnp.float32),
            pltpu.VMEM((C, HALF, cw), jnp.float32),
            pltpu.SemaphoreType.DMA((C,)),
            pltpu.SemaphoreType.DMA((C,)),
            pltpu.SemaphoreType.DMA((C,)),
            pltpu.SemaphoreType.DMA((C,)),
        ],
        compiler_params=pltpu.CompilerParams(collective_id=0),
    )(dy, W)


# device time: 18605 ns/iter; 1.4325x vs baseline; 1.4325x over previous
import jax
import jax.numpy as jnp
from jax import lax
from jax.experimental import pallas as pl
from jax.experimental.pallas import tpu as pltpu

M = 512
HALF = M // 2
C = 16
TR = HALF // C


def kernel(dy, W):
    m, k = dy.shape
    n = W.shape[0]

    def body(dy_ref, w_ref, out_ref, p_ref, q_ref, r1_ref, r2_ref,
             s1, v1, s2, v2):
        my_x = lax.axis_index("x")
        my_y = lax.axis_index("y")
        y_nbr = (my_x, 1 - my_y)
        x_nbr = (1 - my_x, my_y)

        def rdma1(c):
            return pltpu.make_async_remote_copy(
                src_ref=p_ref.at[pl.ds(c * TR, TR), :],
                dst_ref=r1_ref.at[pl.ds(c * TR, TR), :],
                send_sem=s1.at[c], recv_sem=v1.at[c],
                device_id=y_nbr, device_id_type=pl.DeviceIdType.MESH,
            )

        def rdma2(c):
            return pltpu.make_async_remote_copy(
                src_ref=q_ref.at[pl.ds(c * TR, TR), :],
                dst_ref=r2_ref.at[pl.ds(c * TR, TR), :],
                send_sem=s2.at[c], recv_sem=v2.at[c],
                device_id=x_nbr, device_id_type=pl.DeviceIdType.MESH,
            )

        barrier_sem = pltpu.get_barrier_semaphore()
        for nbr in (y_nbr, x_nbr):
            pl.semaphore_signal(
                barrier_sem, inc=1,
                device_id=nbr, device_id_type=pl.DeviceIdType.MESH,
            )

        row0 = my_x * HALF
        p_ref[...] = lax.dot_general(
            dy_ref[pl.ds(row0, HALF), :], w_ref[...],
            dimension_numbers=(((1,), (1,)), ((), ())),
            preferred_element_type=jnp.float32,
        )

        pl.semaphore_wait(barrier_sem, 2)

        for c in range(C):
            rdma1(c).start()

        for c in range(C):
            rdma1(c).wait_recv()
            sl = pl.ds(c * TR, TR)
            q_ref[sl, :] = p_ref[sl, :] + r1_ref[sl, :]
            rdma2(c).start()
            out_ref[pl.ds(row0 + c * TR, TR), :] = q_ref[sl, :]

        for c in range(C):
            rdma2(c).wait_recv()
            out_ref[pl.ds((1 - my_x) * HALF + c * TR, TR), :] = (
                r2_ref[pl.ds(c * TR, TR), :]
            )

        for c in range(C):
            rdma1(c).wait_send()
            rdma2(c).wait_send()

    return pl.pallas_call(
        body,
        out_shape=jax.ShapeDtypeStruct((m, n), jnp.float32),
        in_specs=[
            pl.BlockSpec(memory_space=pltpu.VMEM),
            pl.BlockSpec(memory_space=pltpu.VMEM),
        ],
        out_specs=pl.BlockSpec(memory_space=pltpu.VMEM),
        scratch_shapes=[
            pltpu.VMEM((HALF, n), jnp.float32),
            pltpu.VMEM((HALF, n), jnp.float32),
            pltpu.VMEM((HALF, n), jnp.float32),
            pltpu.VMEM((HALF, n), jnp.float32),
            pltpu.SemaphoreType.DMA((C,)),
            pltpu.SemaphoreType.DMA((C,)),
            pltpu.SemaphoreType.DMA((C,)),
            pltpu.SemaphoreType.DMA((C,)),
        ],
        compiler_params=pltpu.CompilerParams(collective_id=0),
    )(dy, W)
